# baseline (speedup 1.0000x reference)
import functools

import numpy as np
import jax
import jax.numpy as jnp
from jax import lax
from jax.experimental import pallas as pl
from jax.experimental.pallas import tpu as pltpu
from jax.experimental.pallas import tpu_sc as plsc

N_ANC = 80
ROUNDS = 20
NROUT = ROUNDS - 2
NPAIR = N_ANC * (N_ANC - 1) // 2
ROW = N_ANC + NPAIR
LANES = 16
NCHUNK = (NPAIR + LANES - 1) // LANES
NPAD = NCHUNK * LANES
BATCH = 256
NW = 32
BPW = BATCH // NW


def _aux_arrays():
    out = np.zeros((1, NPAD), np.int32)
    p = 0
    for a in range(N_ANC):
        for b in range(a + 1, N_ANC):
            out[0, p] = a * 128 + b
            p += 1
    return out


_AUX = _aux_arrays()


@functools.cache
def _build_sc_forward():
    mesh = plsc.VectorSubcoreMesh(core_axis_name="c", subcore_axis_name="s")

    @functools.partial(
        pl.kernel,
        out_type=jax.ShapeDtypeStruct((BATCH, NROUT * ROW), jnp.float32),
        mesh=mesh,
        compiler_params=pltpu.CompilerParams(needs_layout_passes=False),
        scratch_types=[
            pltpu.VMEM((ROUNDS * N_ANC,), jnp.int32),
            pltpu.VMEM((1, NPAD), jnp.int32),
            pltpu.VMEM((4, NPAD), jnp.float32),
            pltpu.VMEM((N_ANC,), jnp.float32),
            pltpu.VMEM((NPAD * 9,), jnp.float32),
            pltpu.VMEM((N_ANC * 3,), jnp.float32),
            pltpu.VMEM((2 * N_ANC * LANES,), jnp.int32),
            pltpu.VMEM((NROUT * N_ANC,), jnp.int32),
            pltpu.VMEM((NROUT * ROW,), jnp.float32),
        ],
    )
    def _sc_forward(inp_hbm, pnd_hbm, pd_hbm, aux_hbm, out_hbm,
                    inp_v, aux_v, pnd_v, pd_v, lut_v, lutd_v, s3_v, s1_v,
                    outb_v):
        wid = lax.axis_index("s") * 2 + lax.axis_index("c")
        iota = lax.iota(jnp.int32, LANES)
        zeros = jnp.zeros((LANES,), jnp.float32)
        ones = jnp.ones((LANES,), jnp.float32)

        pltpu.sync_copy(aux_hbm, aux_v)
        pltpu.sync_copy(pnd_hbm, pnd_v)
        pltpu.sync_copy(pd_hbm, pd_v)

        def lut_body(c, _):
            o = LANES * c
            p0 = pnd_v[0, pl.ds(o, LANES)]
            p1 = pnd_v[1, pl.ds(o, LANES)]
            p2 = pnd_v[2, pl.ds(o, LANES)]
            p3 = pnd_v[3, pl.ds(o, LANES)]
            f12 = 1.0 / (1.0 + jnp.exp(-p0))
            f9 = f12 / (1.0 + jnp.exp(-p1))
            f8 = f9 / (1.0 + jnp.exp(-p2))
            f6 = f8 / (1.0 + jnp.exp(-p3))
            base = (o + iota) * 9
            for ci, v in enumerate((zeros, f6, f8, f6, f9, f12, f8, f12, ones)):
                plsc.store_scatter(lut_v, [base + ci], v)
            return 0

        lax.fori_loop(0, NCHUNK, lut_body, 0)

        for c in range(N_ANC // LANES):
            o = LANES * c
            sg = 1.0 / (1.0 + jnp.exp(-pd_v[pl.ds(o, LANES)]))
            base = (o + iota) * 3
            plsc.store_scatter(lutd_v, [base], zeros)
            plsc.store_scatter(lutd_v, [base + 1], sg)
            plsc.store_scatter(lutd_v, [base + 2], ones)

        def batch_body(b, _):
            n = wid * BPW + b
            pltpu.sync_copy(inp_hbm.at[n], inp_v)

            init = (tuple(jnp.full((LANES,), -1, jnp.int32) for _ in range(5))
                    + tuple(jnp.full((LANES,), 1, jnp.int32) for _ in range(5)))

            def recur_step(rr, sb, srep, rowb, st, dl):
                for c in range(N_ANC // LANES):
                    o = LANES * c
                    x0 = inp_v[pl.ds(rr * N_ANC + o, LANES)]
                    x1 = inp_v[pl.ds((rr + 1) * N_ANC + o, LANES)]
                    x2 = inp_v[pl.ds((rr + 2) * N_ANC + o, LANES)]
                    de = x0 + x2 - 2 * x0 * x2
                    me = x1 * (1 - (x0 + x2)) + x0 * x2
                    d2 = dl[c] * (1 - 2 * me)
                    s2 = jnp.clip(st[c] + d2 * de, -1, 1)
                    nm = 1 - me
                    st[c] = s2
                    dl[c] = d2 * (1 - s2 * s2 * nm) - s2 * nm
                    s1 = s2 + 1
                    s1_v[pl.ds(sb + o, LANES)] = s1
                    s13 = s1 * 3
                    rep_base = (o + iota) * LANES + srep
                    for l in range(LANES):
                        plsc.store_scatter(s3_v, [rep_base + l], s13)
                    dv = plsc.load_gather(lutd_v, [(o + iota) * 3 + s1])
                    outb_v[pl.ds(rowb + o, LANES)] = dv

            def round_body(t, carry):
                st = list(carry[:5])
                dl = list(carry[5:])
                rr0 = 2 * t
                sb0 = rr0 * N_ANC
                rowb0 = rr0 * ROW
                recur_step(rr0, sb0, 0, rowb0, st, dl)
                recur_step(rr0 + 1, sb0 + N_ANC, N_ANC * LANES,
                           rowb0 + ROW, st, dl)

                nd_base = rowb0 + N_ANC
                rep2 = N_ANC * LANES

                def pair_chunk(o):
                    pij = aux_v[0, pl.ds(o, LANES)]
                    pi = ((pij >> 7) << 4) + iota
                    pj = (pij & 127) + sb0
                    lbase = (o + iota) * 9
                    a3 = plsc.load_gather(s3_v, [pi])
                    b1 = plsc.load_gather(s1_v, [pj])
                    v = plsc.load_gather(lut_v, [lbase + a3 + b1])
                    outb_v[pl.ds(nd_base + o, LANES)] = v
                    a3b = plsc.load_gather(s3_v, [pi + rep2])
                    b1b = plsc.load_gather(s1_v, [pj + N_ANC])
                    vb = plsc.load_gather(lut_v, [lbase + a3b + b1b])
                    outb_v[pl.ds(nd_base + ROW + o, LANES)] = vb

                plsc.parallel_loop(0, NPAIR - LANES // 2, LANES, unroll=8)(
                    pair_chunk)
                pair_chunk(NPAIR - LANES)
                return tuple(st) + tuple(dl)

            lax.fori_loop(0, NROUT // 2, round_body, init)
            pltpu.sync_copy(outb_v, out_hbm.at[n])
            return 0

        lax.fori_loop(0, BPW, batch_body, 0)

    return _sc_forward


def kernel(input, embedding_params_diag, embedding_params_nondiag):
    inp = input.reshape(BATCH, ROUNDS * N_ANC)
    pd = embedding_params_diag.reshape(N_ANC)
    pnd = embedding_params_nondiag.reshape(NPAIR, 4).T
    pnd = jnp.concatenate(
        [pnd, jnp.zeros((4, NPAD - NPAIR), jnp.float32)], axis=1)
    out = _build_sc_forward()(inp, pnd, pd, jnp.asarray(_AUX))
    return out.reshape(BATCH, NROUT, ROW)


# speedup vs baseline: 1.0769x; 1.0769x over previous
import functools

import numpy as np
import jax
import jax.numpy as jnp
from jax import lax
from jax.experimental import pallas as pl
from jax.experimental.pallas import tpu as pltpu
from jax.experimental.pallas import tpu_sc as plsc

N_ANC = 80
ROUNDS = 20
NROUT = ROUNDS - 2
NPAIR = N_ANC * (N_ANC - 1) // 2
ROW = N_ANC + NPAIR
LANES = 16
NCHUNK = (NPAIR + LANES - 1) // LANES
NPAD = NCHUNK * LANES
BATCH = 256
NW = 32
BPW = BATCH // NW


def _aux_arrays():
    out = np.zeros((1, NPAD), np.int32)
    p = 0
    for a in range(N_ANC):
        for b in range(a + 1, N_ANC):
            out[0, p] = a * 128 + b
            p += 1
    return out


_AUX = _aux_arrays()


@functools.cache
def _build_sc_forward():
    mesh = plsc.VectorSubcoreMesh(core_axis_name="c", subcore_axis_name="s")

    @functools.partial(
        pl.kernel,
        out_type=jax.ShapeDtypeStruct((BATCH, NROUT * ROW), jnp.float32),
        mesh=mesh,
        compiler_params=pltpu.CompilerParams(needs_layout_passes=False),
        scratch_types=[
            pltpu.VMEM((ROUNDS * N_ANC,), jnp.int32),
            pltpu.VMEM((1, NPAD), jnp.int32),
            pltpu.VMEM((4, NPAD), jnp.float32),
            pltpu.VMEM((N_ANC,), jnp.float32),
            pltpu.VMEM((NPAD * 9,), jnp.float32),
            pltpu.VMEM((N_ANC * 3,), jnp.float32),
            pltpu.VMEM((NROUT * N_ANC,), jnp.int32),
            pltpu.VMEM((NROUT * N_ANC,), jnp.int32),
            pltpu.VMEM((NROUT * ROW,), jnp.float32),
        ],
    )
    def _sc_forward(inp_hbm, pnd_hbm, pd_hbm, aux_hbm, out_hbm,
                    inp_v, aux_v, pnd_v, pd_v, lut_v, lutd_v, s3_v, s1_v,
                    outb_v):
        wid = lax.axis_index("s") * 2 + lax.axis_index("c")
        iota = lax.iota(jnp.int32, LANES)
        zeros = jnp.zeros((LANES,), jnp.float32)
        ones = jnp.ones((LANES,), jnp.float32)

        pltpu.sync_copy(aux_hbm, aux_v)
        pltpu.sync_copy(pnd_hbm, pnd_v)
        pltpu.sync_copy(pd_hbm, pd_v)

        def lut_body(c, _):
            o = LANES * c
            p0 = pnd_v[0, pl.ds(o, LANES)]
            p1 = pnd_v[1, pl.ds(o, LANES)]
            p2 = pnd_v[2, pl.ds(o, LANES)]
            p3 = pnd_v[3, pl.ds(o, LANES)]
            f12 = 1.0 / (1.0 + jnp.exp(-p0))
            f9 = f12 / (1.0 + jnp.exp(-p1))
            f8 = f9 / (1.0 + jnp.exp(-p2))
            f6 = f8 / (1.0 + jnp.exp(-p3))
            base = (o + iota) * 9
            for ci, v in enumerate((zeros, f6, f8, f6, f9, f12, f8, f12, ones)):
                plsc.store_scatter(lut_v, [base + ci], v)
            return 0

        lax.fori_loop(0, NCHUNK, lut_body, 0)

        for c in range(N_ANC // LANES):
            o = LANES * c
            sg = 1.0 / (1.0 + jnp.exp(-pd_v[pl.ds(o, LANES)]))
            base = (o + iota) * 3
            plsc.store_scatter(lutd_v, [base], zeros)
            plsc.store_scatter(lutd_v, [base + 1], sg)
            plsc.store_scatter(lutd_v, [base + 2], ones)

        def batch_body(b, _):
            n = wid * BPW + b
            pltpu.sync_copy(inp_hbm.at[n], inp_v)

            init = (tuple(jnp.full((LANES,), -1, jnp.int32) for _ in range(5))
                    + tuple(jnp.full((LANES,), 1, jnp.int32) for _ in range(5)))

            def recur_step(rr, sb, rowb, st, dl):
                for c in range(N_ANC // LANES):
                    o = LANES * c
                    x0 = inp_v[pl.ds(rr * N_ANC + o, LANES)]
                    x1 = inp_v[pl.ds((rr + 1) * N_ANC + o, LANES)]
                    x2 = inp_v[pl.ds((rr + 2) * N_ANC + o, LANES)]
                    de = x0 + x2 - 2 * x0 * x2
                    me = x1 * (1 - (x0 + x2)) + x0 * x2
                    d2 = dl[c] * (1 - 2 * me)
                    s2 = jnp.clip(st[c] + d2 * de, -1, 1)
                    nm = 1 - me
                    st[c] = s2
                    dl[c] = d2 * (1 - s2 * s2 * nm) - s2 * nm
                    s1 = s2 + 1
                    s1_v[pl.ds(sb + o, LANES)] = s1
                    s3_v[pl.ds(sb + o, LANES)] = s1 * 3
                    dv = plsc.load_gather(lutd_v, [(o + iota) * 3 + s1])
                    outb_v[pl.ds(rowb + o, LANES)] = dv

            def round_body(t, carry):
                st = list(carry[:5])
                dl = list(carry[5:])
                rr0 = 2 * t
                sb0 = rr0 * N_ANC
                rowb0 = rr0 * ROW
                recur_step(rr0, sb0, rowb0, st, dl)
                recur_step(rr0 + 1, sb0 + N_ANC, rowb0 + ROW, st, dl)

                nd_base = rowb0 + N_ANC

                def pair_chunk(o):
                    pij = aux_v[0, pl.ds(o, LANES)]
                    pi = (pij >> 7) + sb0
                    pj = (pij & 127) + sb0
                    lbase = (o + iota) * 9
                    a3 = plsc.load_gather(s3_v, [pi])
                    b1 = plsc.load_gather(s1_v, [pj])
                    v = plsc.load_gather(lut_v, [lbase + a3 + b1])
                    outb_v[pl.ds(nd_base + o, LANES)] = v
                    a3b = plsc.load_gather(s3_v, [pi + N_ANC])
                    b1b = plsc.load_gather(s1_v, [pj + N_ANC])
                    vb = plsc.load_gather(lut_v, [lbase + a3b + b1b])
                    outb_v[pl.ds(nd_base + ROW + o, LANES)] = vb

                plsc.parallel_loop(0, NPAIR - LANES // 2, LANES, unroll=12)(
                    pair_chunk)
                pair_chunk(NPAIR - LANES)
                return tuple(st) + tuple(dl)

            lax.fori_loop(0, NROUT // 2, round_body, init)
            pltpu.sync_copy(outb_v, out_hbm.at[n])
            return 0

        lax.fori_loop(0, BPW, batch_body, 0)

    return _sc_forward


def kernel(input, embedding_params_diag, embedding_params_nondiag):
    inp = input.reshape(BATCH, ROUNDS * N_ANC)
    pd = embedding_params_diag.reshape(N_ANC)
    pnd = embedding_params_nondiag.reshape(NPAIR, 4).T
    pnd = jnp.concatenate(
        [pnd, jnp.zeros((4, NPAD - NPAIR), jnp.float32)], axis=1)
    out = _build_sc_forward()(inp, pnd, pd, jnp.asarray(_AUX))
    return out.reshape(BATCH, NROUT, ROW)


# speedup vs baseline: 1.0988x; 1.0204x over previous
import functools

import numpy as np
import jax
import jax.numpy as jnp
from jax import lax
from jax.experimental import pallas as pl
from jax.experimental.pallas import tpu as pltpu
from jax.experimental.pallas import tpu_sc as plsc

N_ANC = 80
ROUNDS = 20
NROUT = ROUNDS - 2
NPAIR = N_ANC * (N_ANC - 1) // 2
ROW = N_ANC + NPAIR
LANES = 16
NCHUNK = (NPAIR + LANES - 1) // LANES
NPAD = NCHUNK * LANES
BATCH = 256
NW = 32
BPW = BATCH // NW


def _aux_arrays():
    out = np.zeros((1, NPAD), np.int32)
    p = 0
    for a in range(N_ANC):
        for b in range(a + 1, N_ANC):
            out[0, p] = a * 128 + b
            p += 1
    return out


_AUX = _aux_arrays()


@functools.cache
def _build_sc_forward():
    mesh = plsc.VectorSubcoreMesh(core_axis_name="c", subcore_axis_name="s")

    @functools.partial(
        pl.kernel,
        out_type=jax.ShapeDtypeStruct((BATCH, NROUT * ROW), jnp.float32),
        mesh=mesh,
        compiler_params=pltpu.CompilerParams(needs_layout_passes=False),
        scratch_types=[
            pltpu.VMEM((ROUNDS * N_ANC,), jnp.int32),
            pltpu.VMEM((1, NPAD), jnp.int32),
            pltpu.VMEM((4, NPAD), jnp.float32),
            pltpu.VMEM((N_ANC,), jnp.float32),
            pltpu.VMEM((NPAD * 9,), jnp.float32),
            pltpu.VMEM((N_ANC * 3,), jnp.float32),
            pltpu.VMEM((NROUT * N_ANC,), jnp.int32),
            pltpu.VMEM((NROUT * N_ANC,), jnp.int32),
            pltpu.VMEM((NROUT * ROW,), jnp.float32),
        ],
    )
    def _sc_forward(inp_hbm, pnd_hbm, pd_hbm, aux_hbm, out_hbm,
                    inp_v, aux_v, pnd_v, pd_v, lut_v, lutd_v, s3_v, s1_v,
                    outb_v):
        wid = lax.axis_index("s") * 2 + lax.axis_index("c")
        iota = lax.iota(jnp.int32, LANES)
        zeros = jnp.zeros((LANES,), jnp.float32)
        ones = jnp.ones((LANES,), jnp.float32)

        pltpu.sync_copy(aux_hbm, aux_v)
        pltpu.sync_copy(pnd_hbm, pnd_v)
        pltpu.sync_copy(pd_hbm, pd_v)

        def lut_body(c, _):
            o = LANES * c
            p0 = pnd_v[0, pl.ds(o, LANES)]
            p1 = pnd_v[1, pl.ds(o, LANES)]
            p2 = pnd_v[2, pl.ds(o, LANES)]
            p3 = pnd_v[3, pl.ds(o, LANES)]
            f12 = 1.0 / (1.0 + jnp.exp(-p0))
            f9 = f12 / (1.0 + jnp.exp(-p1))
            f8 = f9 / (1.0 + jnp.exp(-p2))
            f6 = f8 / (1.0 + jnp.exp(-p3))
            base = (o + iota) * 9
            for ci, v in enumerate((zeros, f6, f8, f6, f9, f12, f8, f12, ones)):
                plsc.store_scatter(lut_v, [base + ci], v)
            return 0

        lax.fori_loop(0, NCHUNK, lut_body, 0)

        for c in range(N_ANC // LANES):
            o = LANES * c
            sg = 1.0 / (1.0 + jnp.exp(-pd_v[pl.ds(o, LANES)]))
            base = (o + iota) * 3
            plsc.store_scatter(lutd_v, [base], zeros)
            plsc.store_scatter(lutd_v, [base + 1], sg)
            plsc.store_scatter(lutd_v, [base + 2], ones)

        def batch_body(b, _):
            n = wid * BPW + b
            pltpu.sync_copy(inp_hbm.at[n], inp_v)

            init = (tuple(jnp.full((LANES,), -1, jnp.int32) for _ in range(5))
                    + tuple(jnp.full((LANES,), 1, jnp.int32) for _ in range(5)))

            def recur_step(rr, sb, rowb, st, dl):
                for c in range(N_ANC // LANES):
                    o = LANES * c
                    x0 = inp_v[pl.ds(rr * N_ANC + o, LANES)]
                    x1 = inp_v[pl.ds((rr + 1) * N_ANC + o, LANES)]
                    x2 = inp_v[pl.ds((rr + 2) * N_ANC + o, LANES)]
                    de = x0 + x2 - 2 * x0 * x2
                    me = x1 * (1 - (x0 + x2)) + x0 * x2
                    d2 = dl[c] * (1 - 2 * me)
                    s2 = jnp.clip(st[c] + d2 * de, -1, 1)
                    nm = 1 - me
                    st[c] = s2
                    dl[c] = d2 * (1 - s2 * s2 * nm) - s2 * nm
                    s1 = s2 + 1
                    s1_v[pl.ds(sb + o, LANES)] = s1
                    s3_v[pl.ds(sb + o, LANES)] = s1 * 3
                    dv = plsc.load_gather(lutd_v, [(o + iota) * 3 + s1])
                    outb_v[pl.ds(rowb + o, LANES)] = dv

            def round_body(t, carry):
                st = list(carry[:5])
                dl = list(carry[5:])
                rr0 = 2 * t
                sb0 = rr0 * N_ANC
                rowb0 = rr0 * ROW
                recur_step(rr0, sb0, rowb0, st, dl)
                recur_step(rr0 + 1, sb0 + N_ANC, rowb0 + ROW, st, dl)

                nd_base = rowb0 + N_ANC

                def pair_chunk(o):
                    pij = aux_v[0, pl.ds(o, LANES)]
                    pi = (pij >> 7) + sb0
                    pj = (pij & 127) + sb0
                    lbase = (o + iota) * 9
                    a3 = plsc.load_gather(s3_v, [pi])
                    b1 = plsc.load_gather(s1_v, [pj])
                    v = plsc.load_gather(lut_v, [lbase + a3 + b1])
                    outb_v[pl.ds(nd_base + o, LANES)] = v
                    a3b = plsc.load_gather(s3_v, [pi + N_ANC])
                    b1b = plsc.load_gather(s1_v, [pj + N_ANC])
                    vb = plsc.load_gather(lut_v, [lbase + a3b + b1b])
                    outb_v[pl.ds(nd_base + ROW + o, LANES)] = vb

                plsc.parallel_loop(0, NPAIR - LANES // 2, LANES, unroll=8)(
                    pair_chunk)
                pair_chunk(NPAIR - LANES)
                return tuple(st) + tuple(dl)

            lax.fori_loop(0, NROUT // 2, round_body, init)
            pltpu.sync_copy(outb_v, out_hbm.at[n])
            return 0

        lax.fori_loop(0, BPW, batch_body, 0)

    return _sc_forward


def kernel(input, embedding_params_diag, embedding_params_nondiag):
    inp = input.reshape(BATCH, ROUNDS * N_ANC)
    pd = embedding_params_diag.reshape(N_ANC)
    pnd = embedding_params_nondiag.reshape(NPAIR, 4).T
    pnd = jnp.concatenate(
        [pnd, jnp.zeros((4, NPAD - NPAIR), jnp.float32)], axis=1)
    out = _build_sc_forward()(inp, pnd, pd, jnp.asarray(_AUX))
    return out.reshape(BATCH, NROUT, ROW)
